# baseline (speedup 1.0000x reference)
import jax
import jax.numpy as jnp
from jax import lax
from jax.experimental import pallas as pl
from jax.experimental.pallas import tpu as pltpu
from jax.experimental.pallas import tpu_sc as plsc

NUM_CORES = 2
NUM_SUBCORES = 16
NW = NUM_CORES * NUM_SUBCORES
LANES = 16
CHUNK = 128
AUG = 128
NFEAT = 64


def _cf_score_kernel(aug_user, aug_item, user_id, pos_item_id, neg_item_id,
                     out_hbm,
                     idx_u, idx_p, idx_n,
                     u_b0, u_b1, p_b0, p_b1, n_b0, n_b1,
                     po_v, no_v,
                     sem_u0, sem_u1, sem_p0, sem_p1, sem_n0, sem_n1):
    b = user_id.shape[0]
    bw = b // NW
    nch = bw // CHUNK

    wid = lax.axis_index("s") * NUM_CORES + lax.axis_index("c")
    base = wid * bw

    pltpu.sync_copy(user_id.at[pl.ds(base, bw)], idx_u)
    pltpu.sync_copy(pos_item_id.at[pl.ds(base, bw)], idx_p)
    pltpu.sync_copy(neg_item_id.at[pl.ds(base, bw)], idx_n)

    ubufs = (u_b0, u_b1)
    pbufs = (p_b0, p_b1)
    nbufs = (n_b0, n_b1)
    usems = (sem_u0, sem_u1)
    psems = (sem_p0, sem_p1)
    nsems = (sem_n0, sem_n1)

    def fire(ci):
        par = ci % 2
        sl = pl.ds(ci * CHUNK, CHUNK)
        return (
            pltpu.async_copy(aug_user.at[idx_u.at[sl]], ubufs[par], usems[par]),
            pltpu.async_copy(aug_item.at[idx_p.at[sl]], pbufs[par], psems[par]),
            pltpu.async_copy(aug_item.at[idx_n.at[sl]], nbufs[par], nsems[par]),
        )

    lane_iota = lax.iota(jnp.int32, LANES)
    feat_mask = jnp.full((LANES,), NFEAT - 1, jnp.int32)
    bias_col = jnp.full((LANES,), NFEAT, jnp.int32)

    inflight = fire(0)
    for ci in range(nch):
        nxt = fire(ci + 1) if ci + 1 < nch else ()
        for cp in inflight:
            cp.wait()
        inflight = nxt

        ub, pb, nb = ubufs[ci % 2], pbufs[ci % 2], nbufs[ci % 2]
        out0 = ci * CHUNK

        def body(g, carry, ub=ub, pb=pb, nb=nb, out0=out0):
            row0 = g * LANES
            rows = row0 + lane_iota
            accp = jnp.zeros((LANES,), jnp.float32)
            accn = jnp.zeros((LANES,), jnp.float32)
            for fc in range(NFEAT):
                colf = (lane_iota + fc) & feat_mask
                uv = plsc.load_gather(ub, [rows, colf])
                pv = plsc.load_gather(pb, [rows, colf])
                nv = plsc.load_gather(nb, [rows, colf])
                accp = accp + uv * pv
                accn = accn + uv * nv
            ubv = plsc.load_gather(ub, [rows, bias_col])
            pbv = plsc.load_gather(pb, [rows, bias_col])
            nbv = plsc.load_gather(nb, [rows, bias_col])
            po_v[pl.ds(out0 + row0, LANES)] = accp + ubv + pbv
            no_v[pl.ds(out0 + row0, LANES)] = accn + ubv + nbv
            return carry

        lax.fori_loop(0, CHUNK // LANES, body, 0)

    pltpu.sync_copy(po_v, out_hbm.at[0, pl.ds(base, bw)])
    pltpu.sync_copy(no_v, out_hbm.at[1, pl.ds(base, bw)])


@jax.jit
def kernel(user_emb, item_emb, user_bias, item_bias, user_id, pos_item_id, neg_item_id):
    b = user_id.shape[0]
    bw = b // NW
    n_users = user_emb.shape[0]
    n_items = item_emb.shape[0]
    f = user_emb.shape[1]
    pad = AUG - f - 1
    aug_user = jnp.concatenate(
        [user_emb, user_bias[:, None],
         jnp.zeros((n_users, pad), jnp.float32)], axis=1)
    aug_item = jnp.concatenate(
        [item_emb, item_bias[:, None],
         jnp.zeros((n_items, pad), jnp.float32)], axis=1)

    mesh = plsc.VectorSubcoreMesh(
        core_axis_name="c", subcore_axis_name="s",
        num_cores=NUM_CORES, num_subcores=NUM_SUBCORES)
    run = pl.kernel(
        _cf_score_kernel,
        out_type=jax.ShapeDtypeStruct((2, b), jnp.float32),
        mesh=mesh,
        compiler_params=pltpu.CompilerParams(
            needs_layout_passes=False, use_tc_tiling_on_sc=True),
        scratch_types=[
            pltpu.VMEM((bw,), jnp.int32),
            pltpu.VMEM((bw,), jnp.int32),
            pltpu.VMEM((bw,), jnp.int32),
            pltpu.VMEM((CHUNK, AUG), jnp.float32),
            pltpu.VMEM((CHUNK, AUG), jnp.float32),
            pltpu.VMEM((CHUNK, AUG), jnp.float32),
            pltpu.VMEM((CHUNK, AUG), jnp.float32),
            pltpu.VMEM((CHUNK, AUG), jnp.float32),
            pltpu.VMEM((CHUNK, AUG), jnp.float32),
            pltpu.VMEM((bw,), jnp.float32),
            pltpu.VMEM((bw,), jnp.float32),
            pltpu.SemaphoreType.DMA,
            pltpu.SemaphoreType.DMA,
            pltpu.SemaphoreType.DMA,
            pltpu.SemaphoreType.DMA,
            pltpu.SemaphoreType.DMA,
            pltpu.SemaphoreType.DMA,
        ],
    )
    return run(aug_user, aug_item, user_id, pos_item_id, neg_item_id)


# speedup vs baseline: 1.4734x; 1.4734x over previous
import jax
import jax.numpy as jnp
from jax import lax
from jax.experimental import pallas as pl
from jax.experimental.pallas import tpu as pltpu
from jax.experimental.pallas import tpu_sc as plsc

NUM_CORES = 2
NUM_SUBCORES = 16
NW = NUM_CORES * NUM_SUBCORES
LANES = 16
CHUNK = 128
NFEAT = 64


def _cf_score_kernel(user_emb, item_emb, user_bias2, item_bias2,
                     user_id, pos_item_id, neg_item_id, out_hbm,
                     idx_u, idx_p, idx_n, idxb_u, idxb_p, idxb_n,
                     u_b0, u_b1, p_b0, p_b1, n_b0, n_b1,
                     ub_b0, ub_b1, pb_b0, pb_b1, nb_b0, nb_b1,
                     po_v, no_v,
                     sem_u0, sem_u1, sem_p0, sem_p1, sem_n0, sem_n1):
    b = user_id.shape[0]
    bw = b // NW
    nch = bw // CHUNK

    wid = lax.axis_index("s") * NUM_CORES + lax.axis_index("c")
    base = wid * bw

    pltpu.sync_copy(user_id.at[pl.ds(base, bw)], idx_u)
    pltpu.sync_copy(pos_item_id.at[pl.ds(base, bw)], idx_p)
    pltpu.sync_copy(neg_item_id.at[pl.ds(base, bw)], idx_n)

    for j in range(bw // LANES):
        sl = pl.ds(j * LANES, LANES)
        idxb_u[sl] = lax.shift_right_logical(idx_u[sl], 4)
        idxb_p[sl] = lax.shift_right_logical(idx_p[sl], 4)
        idxb_n[sl] = lax.shift_right_logical(idx_n[sl], 4)

    ubufs, pbufs, nbufs = (u_b0, u_b1), (p_b0, p_b1), (n_b0, n_b1)
    ubb, pbb, nbb = (ub_b0, ub_b1), (pb_b0, pb_b1), (nb_b0, nb_b1)
    usems, psems, nsems = (sem_u0, sem_u1), (sem_p0, sem_p1), (sem_n0, sem_n1)

    def fire(ci):
        par = ci % 2
        sl = pl.ds(ci * CHUNK, CHUNK)
        return (
            pltpu.async_copy(user_emb.at[idx_u.at[sl]], ubufs[par], usems[par]),
            pltpu.async_copy(item_emb.at[idx_p.at[sl]], pbufs[par], psems[par]),
            pltpu.async_copy(item_emb.at[idx_n.at[sl]], nbufs[par], nsems[par]),
            pltpu.async_copy(user_bias2.at[idxb_u.at[sl]], ubb[par], usems[par]),
            pltpu.async_copy(item_bias2.at[idxb_p.at[sl]], pbb[par], psems[par]),
            pltpu.async_copy(item_bias2.at[idxb_n.at[sl]], nbb[par], nsems[par]),
        )

    lane_iota = lax.iota(jnp.int32, LANES)
    feat_mask = jnp.full((LANES,), NFEAT - 1, jnp.int32)
    lane_mask = jnp.full((LANES,), LANES - 1, jnp.int32)

    inflight = fire(0)
    for ci in range(nch):
        nxt = fire(ci + 1) if ci + 1 < nch else ()
        for cp in inflight:
            cp.wait()
        inflight = nxt

        par = ci % 2
        ub, pb, nb = ubufs[par], pbufs[par], nbufs[par]
        ubr, pbr, nbr = ubb[par], pbb[par], nbb[par]
        out0 = ci * CHUNK

        def body(g, carry, ub=ub, pb=pb, nb=nb, ubr=ubr, pbr=pbr, nbr=nbr,
                 out0=out0):
            row0 = g * LANES
            rows = row0 + lane_iota
            accp = jnp.zeros((LANES,), jnp.float32)
            accn = jnp.zeros((LANES,), jnp.float32)
            for fc in range(NFEAT):
                colf = (lane_iota + fc) & feat_mask
                uv = plsc.load_gather(ub, [rows, colf])
                pv = plsc.load_gather(pb, [rows, colf])
                nv = plsc.load_gather(nb, [rows, colf])
                accp = accp + uv * pv
                accn = accn + uv * nv
            gsl = pl.ds(out0 + row0, LANES)
            ubv = plsc.load_gather(ubr, [rows, idx_u[gsl] & lane_mask])
            pbv = plsc.load_gather(pbr, [rows, idx_p[gsl] & lane_mask])
            nbv = plsc.load_gather(nbr, [rows, idx_n[gsl] & lane_mask])
            po_v[gsl] = accp + ubv + pbv
            no_v[gsl] = accn + ubv + nbv
            return carry

        lax.fori_loop(0, CHUNK // LANES, body, 0)

    pltpu.sync_copy(po_v, out_hbm.at[0, pl.ds(base, bw)])
    pltpu.sync_copy(no_v, out_hbm.at[1, pl.ds(base, bw)])


@jax.jit
def kernel(user_emb, item_emb, user_bias, item_bias, user_id, pos_item_id, neg_item_id):
    b = user_id.shape[0]
    bw = b // NW
    f = user_emb.shape[1]
    user_bias2 = user_bias.reshape(-1, LANES)
    item_bias2 = item_bias.reshape(-1, LANES)
    mesh = plsc.VectorSubcoreMesh(
        core_axis_name="c", subcore_axis_name="s",
        num_cores=NUM_CORES, num_subcores=NUM_SUBCORES)
    run = pl.kernel(
        _cf_score_kernel,
        out_type=jax.ShapeDtypeStruct((2, b), jnp.float32),
        mesh=mesh,
        compiler_params=pltpu.CompilerParams(
            needs_layout_passes=False, use_tc_tiling_on_sc=False,
            disable_bounds_checks=True, disable_semaphore_checks=True),
        scratch_types=[
            pltpu.VMEM((bw,), jnp.int32),
            pltpu.VMEM((bw,), jnp.int32),
            pltpu.VMEM((bw,), jnp.int32),
            pltpu.VMEM((bw,), jnp.int32),
            pltpu.VMEM((bw,), jnp.int32),
            pltpu.VMEM((bw,), jnp.int32),
            pltpu.VMEM((CHUNK, f), jnp.float32),
            pltpu.VMEM((CHUNK, f), jnp.float32),
            pltpu.VMEM((CHUNK, f), jnp.float32),
            pltpu.VMEM((CHUNK, f), jnp.float32),
            pltpu.VMEM((CHUNK, f), jnp.float32),
            pltpu.VMEM((CHUNK, f), jnp.float32),
            pltpu.VMEM((CHUNK, LANES), jnp.float32),
            pltpu.VMEM((CHUNK, LANES), jnp.float32),
            pltpu.VMEM((CHUNK, LANES), jnp.float32),
            pltpu.VMEM((CHUNK, LANES), jnp.float32),
            pltpu.VMEM((CHUNK, LANES), jnp.float32),
            pltpu.VMEM((CHUNK, LANES), jnp.float32),
            pltpu.VMEM((bw,), jnp.float32),
            pltpu.VMEM((bw,), jnp.float32),
            pltpu.SemaphoreType.DMA,
            pltpu.SemaphoreType.DMA,
            pltpu.SemaphoreType.DMA,
            pltpu.SemaphoreType.DMA,
            pltpu.SemaphoreType.DMA,
            pltpu.SemaphoreType.DMA,
        ],
    )
    return run(user_emb, item_emb, user_bias2, item_bias2,
               user_id, pos_item_id, neg_item_id)


# speedup vs baseline: 1.4750x; 1.0011x over previous
import jax
import jax.numpy as jnp
from jax import lax
from jax.experimental import pallas as pl
from jax.experimental.pallas import tpu as pltpu
from jax.experimental.pallas import tpu_sc as plsc

NUM_CORES = 2
NUM_SUBCORES = 16
NW = NUM_CORES * NUM_SUBCORES
LANES = 16
CHUNK = 128
NFEAT = 64


def _cf_score_kernel(user_emb, item_emb, user_bias2, item_bias2,
                     user_id, pos_item_id, neg_item_id, out_hbm,
                     idx_u, idx_p, idx_n, idxb_u, idxb_p, idxb_n,
                     u_b0, u_b1, p_b0, p_b1, n_b0, n_b1,
                     ub_b0, ub_b1, pb_b0, pb_b1, nb_b0, nb_b1,
                     po_v, no_v,
                     sem_u0, sem_u1, sem_p0, sem_p1, sem_n0, sem_n1):
    b = user_id.shape[0]
    bw = b // NW
    nch = bw // CHUNK

    wid = lax.axis_index("s") * NUM_CORES + lax.axis_index("c")
    base = wid * bw

    pltpu.sync_copy(user_id.at[pl.ds(base, bw)], idx_u)
    pltpu.sync_copy(pos_item_id.at[pl.ds(base, bw)], idx_p)
    pltpu.sync_copy(neg_item_id.at[pl.ds(base, bw)], idx_n)

    for j in range(bw // LANES):
        sl = pl.ds(j * LANES, LANES)
        idxb_u[sl] = lax.shift_right_logical(idx_u[sl], 4)
        idxb_p[sl] = lax.shift_right_logical(idx_p[sl], 4)
        idxb_n[sl] = lax.shift_right_logical(idx_n[sl], 4)

    ubufs, pbufs, nbufs = (u_b0, u_b1), (p_b0, p_b1), (n_b0, n_b1)
    ubb, pbb, nbb = (ub_b0, ub_b1), (pb_b0, pb_b1), (nb_b0, nb_b1)
    usems, psems, nsems = (sem_u0, sem_u1), (sem_p0, sem_p1), (sem_n0, sem_n1)

    def fire(ci):
        par = ci % 2
        sl = pl.ds(ci * CHUNK, CHUNK)
        return (
            pltpu.async_copy(user_emb.at[idx_u.at[sl]], ubufs[par], usems[par]),
            pltpu.async_copy(item_emb.at[idx_p.at[sl]], pbufs[par], psems[par]),
            pltpu.async_copy(item_emb.at[idx_n.at[sl]], nbufs[par], nsems[par]),
            pltpu.async_copy(user_bias2.at[idxb_u.at[sl]], ubb[par], usems[par]),
            pltpu.async_copy(item_bias2.at[idxb_p.at[sl]], pbb[par], psems[par]),
            pltpu.async_copy(item_bias2.at[idxb_n.at[sl]], nbb[par], nsems[par]),
        )

    lane_iota = lax.iota(jnp.int32, LANES)
    feat_mask = jnp.full((LANES,), NFEAT - 1, jnp.int32)
    lane_mask = jnp.full((LANES,), LANES - 1, jnp.int32)

    inflight = fire(0)
    for ci in range(nch):
        nxt = fire(ci + 1) if ci + 1 < nch else ()
        for cp in inflight:
            cp.wait()
        inflight = nxt

        par = ci % 2
        ub, pb, nb = ubufs[par], pbufs[par], nbufs[par]
        ubr, pbr, nbr = ubb[par], pbb[par], nbb[par]
        out0 = ci * CHUNK

        def body(g, carry, ub=ub, pb=pb, nb=nb, ubr=ubr, pbr=pbr, nbr=nbr,
                 out0=out0):
            row0 = g * LANES
            rows = row0 + lane_iota
            accp = jnp.zeros((LANES,), jnp.float32)
            accn = jnp.zeros((LANES,), jnp.float32)
            for fc in range(NFEAT):
                colf = (lane_iota + fc) & feat_mask
                uv = plsc.load_gather(ub, [rows, colf])
                pv = plsc.load_gather(pb, [rows, colf])
                nv = plsc.load_gather(nb, [rows, colf])
                accp = accp + uv * pv
                accn = accn + uv * nv
            gsl = pl.ds(out0 + row0, LANES)
            ubv = plsc.load_gather(ubr, [rows, idx_u[gsl] & lane_mask])
            pbv = plsc.load_gather(pbr, [rows, idx_p[gsl] & lane_mask])
            nbv = plsc.load_gather(nbr, [rows, idx_n[gsl] & lane_mask])
            po_v[gsl] = accp + ubv + pbv
            no_v[gsl] = accn + ubv + nbv
            return carry

        lax.fori_loop(0, CHUNK // LANES, body, 0)

    pltpu.sync_copy(po_v, out_hbm.at[0, pl.ds(base, bw)])
    pltpu.sync_copy(no_v, out_hbm.at[1, pl.ds(base, bw)])


@jax.jit
def kernel(user_emb, item_emb, user_bias, item_bias, user_id, pos_item_id, neg_item_id):
    b = user_id.shape[0]
    bw = b // NW
    f = user_emb.shape[1]
    user_bias2 = user_bias.reshape(-1, LANES)
    item_bias2 = item_bias.reshape(-1, LANES)
    mesh = plsc.VectorSubcoreMesh(
        core_axis_name="c", subcore_axis_name="s",
        num_cores=NUM_CORES, num_subcores=NUM_SUBCORES)
    run = pl.kernel(
        _cf_score_kernel,
        out_type=jax.ShapeDtypeStruct((2, b), jnp.float32),
        mesh=mesh,
        compiler_params=pltpu.CompilerParams(
            needs_layout_passes=False, use_tc_tiling_on_sc=False,
            disable_bounds_checks=True, disable_semaphore_checks=True,
            skip_device_barrier=True),
        scratch_types=[
            pltpu.VMEM((bw,), jnp.int32),
            pltpu.VMEM((bw,), jnp.int32),
            pltpu.VMEM((bw,), jnp.int32),
            pltpu.VMEM((bw,), jnp.int32),
            pltpu.VMEM((bw,), jnp.int32),
            pltpu.VMEM((bw,), jnp.int32),
            pltpu.VMEM((CHUNK, f), jnp.float32),
            pltpu.VMEM((CHUNK, f), jnp.float32),
            pltpu.VMEM((CHUNK, f), jnp.float32),
            pltpu.VMEM((CHUNK, f), jnp.float32),
            pltpu.VMEM((CHUNK, f), jnp.float32),
            pltpu.VMEM((CHUNK, f), jnp.float32),
            pltpu.VMEM((CHUNK, LANES), jnp.float32),
            pltpu.VMEM((CHUNK, LANES), jnp.float32),
            pltpu.VMEM((CHUNK, LANES), jnp.float32),
            pltpu.VMEM((CHUNK, LANES), jnp.float32),
            pltpu.VMEM((CHUNK, LANES), jnp.float32),
            pltpu.VMEM((CHUNK, LANES), jnp.float32),
            pltpu.VMEM((bw,), jnp.float32),
            pltpu.VMEM((bw,), jnp.float32),
            pltpu.SemaphoreType.DMA,
            pltpu.SemaphoreType.DMA,
            pltpu.SemaphoreType.DMA,
            pltpu.SemaphoreType.DMA,
            pltpu.SemaphoreType.DMA,
            pltpu.SemaphoreType.DMA,
        ],
    )
    return run(user_emb, item_emb, user_bias2, item_bias2,
               user_id, pos_item_id, neg_item_id)


# speedup vs baseline: 1.6223x; 1.0999x over previous
import jax
import jax.numpy as jnp
from jax import lax
from jax.experimental import pallas as pl
from jax.experimental.pallas import tpu as pltpu
from jax.experimental.pallas import tpu_sc as plsc

NUM_CORES = 2
NUM_SUBCORES = 16
NW = NUM_CORES * NUM_SUBCORES
LANES = 16
CHUNK = 128
NFEAT = 64


def _cf_score_kernel(user_emb, item_emb, user_bias, item_bias,
                     user_id, pos_item_id, neg_item_id, out_hbm,
                     idx_u, idx_p, idx_n,
                     u_b0, u_b1, p_b0, p_b1, n_b0, n_b1,
                     ub_b0, ub_b1, pb_b0, pb_b1, nb_b0, nb_b1,
                     po_v, no_v,
                     sem_u0, sem_u1, sem_p0, sem_p1, sem_n0, sem_n1):
    b = user_id.shape[0]
    bw = b // NW
    nch = bw // CHUNK

    wid = lax.axis_index("s") * NUM_CORES + lax.axis_index("c")
    base = wid * bw

    pltpu.sync_copy(user_id.at[pl.ds(base, bw)], idx_u)
    pltpu.sync_copy(pos_item_id.at[pl.ds(base, bw)], idx_p)
    pltpu.sync_copy(neg_item_id.at[pl.ds(base, bw)], idx_n)

    ubufs, pbufs, nbufs = (u_b0, u_b1), (p_b0, p_b1), (n_b0, n_b1)
    ubb, pbb, nbb = (ub_b0, ub_b1), (pb_b0, pb_b1), (nb_b0, nb_b1)
    usems, psems, nsems = (sem_u0, sem_u1), (sem_p0, sem_p1), (sem_n0, sem_n1)

    def fire(ci):
        par = ci % 2
        sl = pl.ds(ci * CHUNK, CHUNK)
        return (
            pltpu.async_copy(user_emb.at[idx_u.at[sl]], ubufs[par], usems[par]),
            pltpu.async_copy(item_emb.at[idx_p.at[sl]], pbufs[par], psems[par]),
            pltpu.async_copy(item_emb.at[idx_n.at[sl]], nbufs[par], nsems[par]),
            pltpu.async_copy(user_bias.at[idx_u.at[sl]], ubb[par], usems[par]),
            pltpu.async_copy(item_bias.at[idx_p.at[sl]], pbb[par], psems[par]),
            pltpu.async_copy(item_bias.at[idx_n.at[sl]], nbb[par], nsems[par]),
        )

    lane_iota = lax.iota(jnp.int32, LANES)
    feat_mask = jnp.full((LANES,), NFEAT - 1, jnp.int32)

    inflight = fire(0)
    for ci in range(nch):
        nxt = fire(ci + 1) if ci + 1 < nch else ()
        for cp in inflight:
            cp.wait()
        inflight = nxt

        par = ci % 2
        ub, pb, nb = ubufs[par], pbufs[par], nbufs[par]
        ubr, pbr, nbr = ubb[par], pbb[par], nbb[par]
        out0 = ci * CHUNK

        def body(g, carry, ub=ub, pb=pb, nb=nb, ubr=ubr, pbr=pbr, nbr=nbr,
                 out0=out0):
            row0 = g * LANES
            rows = row0 + lane_iota
            accp = jnp.zeros((LANES,), jnp.float32)
            accn = jnp.zeros((LANES,), jnp.float32)
            for fc in range(NFEAT):
                colf = (lane_iota + fc) & feat_mask
                uv = plsc.load_gather(ub, [rows, colf])
                pv = plsc.load_gather(pb, [rows, colf])
                nv = plsc.load_gather(nb, [rows, colf])
                accp = accp + uv * pv
                accn = accn + uv * nv
            lsl = pl.ds(row0, LANES)
            ubv = ubr[lsl]
            po_v[pl.ds(out0 + row0, LANES)] = accp + ubv + pbr[lsl]
            no_v[pl.ds(out0 + row0, LANES)] = accn + ubv + nbr[lsl]
            return carry

        lax.fori_loop(0, CHUNK // LANES, body, 0)

    pltpu.sync_copy(po_v, out_hbm.at[0, pl.ds(base, bw)])
    pltpu.sync_copy(no_v, out_hbm.at[1, pl.ds(base, bw)])


@jax.jit
def kernel(user_emb, item_emb, user_bias, item_bias, user_id, pos_item_id, neg_item_id):
    b = user_id.shape[0]
    bw = b // NW
    f = user_emb.shape[1]
    mesh = plsc.VectorSubcoreMesh(
        core_axis_name="c", subcore_axis_name="s",
        num_cores=NUM_CORES, num_subcores=NUM_SUBCORES)
    run = pl.kernel(
        _cf_score_kernel,
        out_type=jax.ShapeDtypeStruct((2, b), jnp.float32),
        mesh=mesh,
        compiler_params=pltpu.CompilerParams(
            needs_layout_passes=False, use_tc_tiling_on_sc=False,
            disable_bounds_checks=True, disable_semaphore_checks=True),
        scratch_types=[
            pltpu.VMEM((bw,), jnp.int32),
            pltpu.VMEM((bw,), jnp.int32),
            pltpu.VMEM((bw,), jnp.int32),
            pltpu.VMEM((CHUNK, f), jnp.float32),
            pltpu.VMEM((CHUNK, f), jnp.float32),
            pltpu.VMEM((CHUNK, f), jnp.float32),
            pltpu.VMEM((CHUNK, f), jnp.float32),
            pltpu.VMEM((CHUNK, f), jnp.float32),
            pltpu.VMEM((CHUNK, f), jnp.float32),
            pltpu.VMEM((CHUNK,), jnp.float32),
            pltpu.VMEM((CHUNK,), jnp.float32),
            pltpu.VMEM((CHUNK,), jnp.float32),
            pltpu.VMEM((CHUNK,), jnp.float32),
            pltpu.VMEM((CHUNK,), jnp.float32),
            pltpu.VMEM((CHUNK,), jnp.float32),
            pltpu.VMEM((bw,), jnp.float32),
            pltpu.VMEM((bw,), jnp.float32),
            pltpu.SemaphoreType.DMA,
            pltpu.SemaphoreType.DMA,
            pltpu.SemaphoreType.DMA,
            pltpu.SemaphoreType.DMA,
            pltpu.SemaphoreType.DMA,
            pltpu.SemaphoreType.DMA,
        ],
    )
    return run(user_emb, item_emb, user_bias, item_bias,
               user_id, pos_item_id, neg_item_id)
